# Initial kernel scaffold; baseline (speedup 1.0000x reference)
import jax
import jax.numpy as jnp
from jax.experimental import pallas as pl


def kernel(mask, match, context, geometric, context_anchor_points, geometric_anchor_points, ln_match_g, ln_match_b, ln_ctx_g, ln_ctx_b, ln_geo_g, ln_geo_b, w_wctx, b_wctx, w_wgeo, b_wgeo, w_vctx, b_vctx, w_vgeo, b_vgeo, w_kpctx, b_kpctx, w_kpgeo, b_kpgeo, w_out, b_out):
    raise NotImplementedError("write your pallas kernel here")


# speedup vs baseline: 36.4682x; 36.4682x over previous
import functools

import jax
import jax.numpy as jnp
from jax import lax
from jax.experimental import pallas as pl
from jax.experimental.pallas import tpu as pltpu
from jax.experimental.pallas import tpu_sc as plsc

B = 2
C = 128
H = 96
W = 96
P = 9
G = 8
CG = C // G
HW = H * W
NPTS = 4 * P
NCW = NPTS * G
NQ = 2 * B * HW

TQ = 512
NT = HW // TQ

NCORES = 2
NSUB = 16
NW = NCORES * NSUB
QPW = NQ // NW
SCQ = 64
NSC = QPW // SCQ
GRP = 8
NGRP = SCQ // GRP

_HIGH = jax.lax.Precision.HIGHEST


def _dotT(a, w):
    return lax.dot_general(a, w, (((1,), (1,)), ((), ())),
                           preferred_element_type=jnp.float32)


def _ln(x, g, b):
    mu = jnp.mean(x, axis=-1, keepdims=True)
    var = jnp.mean((x - mu) ** 2, axis=-1, keepdims=True)
    return (x - mu) * lax.rsqrt(var + 1e-5) * g + b


def _prep_body(xm, xc, xg, cax, cay, gax, gay,
               lmg, lmb, lcg, lcb, lgg, lgb,
               wvc, bvc, wvg, bvg, wwc, bwc, wwg, bwg,
               wkcx, bkcx, wkcy, bkcy, wkgx, bkgx, wkgy, bkgy,
               o_vtab, o_cw, o_idx,
               o_kxc, o_kyc, o_kxg, o_kyg):
    b = pl.program_id(0)
    mn = _ln(xm[0], lmg[...], lmb[...])
    cn = _ln(xc[0], lcg[...], lcb[...])
    gn = _ln(xg[0], lgg[...], lgb[...])

    o_vtab[0, 0] = _dotT(cn, wvc[...]) + bvc[...]
    o_vtab[1, 0] = _dotT(gn, wvg[...]) + bvg[...]

    rep = (lax.broadcasted_iota(jnp.int32, (NPTS, NCW), 1) // G
           == lax.broadcasted_iota(jnp.int32, (NPTS, NCW), 0)
           ).astype(jnp.float32)

    for tsel, (ww, bw, wkx, bkx, wky, bky, ax, ay, o_kx, o_ky) in enumerate((
            (wwc, bwc, wkcx, bkcx, wkcy, bkcy, cax, cay, o_kxc, o_kyc),
            (wwg, bwg, wkgx, bkgx, wkgy, bkgy, gax, gay, o_kxg, o_kyg))):
        logits = _dotT(mn, ww[...]) + bw[...]
        sl = [logits[:, p * G:(p + 1) * G] for p in range(P)]
        m = sl[0]
        for p in range(1, P):
            m = jnp.maximum(m, sl[p])
        es = [jnp.exp(s - m) for s in sl]
        tot = es[0]
        for p in range(1, P):
            tot = tot + es[p]
        inv = 1.0 / tot
        gw = jnp.concatenate([e * inv for e in es], axis=1)

        kx = ax[0] + _dotT(mn, wkx[...]) + bkx[...]
        ky = ay[0] + _dotT(mn, wky[...]) + bky[...]
        o_kx[0] = kx
        o_ky[0] = ky
        x = kx * W - 0.5
        y = ky * H - 0.5
        x0 = jnp.floor(x)
        y0 = jnp.floor(y)
        wx1 = x - x0
        wx0 = 1.0 - wx1
        wy1 = y - y0
        wy0 = 1.0 - wy1
        idx_cols = []
        bw_cols = []
        for dx, dy, wxs, wys in ((0, 0, wx0, wy0), (1, 0, wx1, wy0),
                                 (0, 1, wx0, wy1), (1, 1, wx1, wy1)):
            xi = x0 + dx
            yi = y0 + dy
            valid = ((xi >= 0.0) & (xi <= W - 1.0)
                     & (yi >= 0.0) & (yi <= H - 1.0))
            xcl = jnp.clip(xi, 0.0, W - 1.0).astype(jnp.int32)
            ycl = jnp.clip(yi, 0.0, H - 1.0).astype(jnp.int32)
            idx_cols.append(ycl * W + xcl + (tsel * B + b) * HW)
            bw_cols.append(wxs * wys * valid.astype(jnp.float32))
        o_idx[tsel, 0] = jnp.concatenate(idx_cols, axis=1)
        bwm = jnp.concatenate(bw_cols, axis=1)
        o_cw[tsel, 0] = (
            lax.dot_general(bwm, rep, (((1,), (0,)), ((), ())),
                            precision=_HIGH,
                            preferred_element_type=jnp.float32)
            * jnp.concatenate([gw, gw, gw, gw], axis=1))


def _tc_prep(xm, xc, xg, cax, cay, gax, gay,
             lmg, lmb, lcg, lcb, lgg, lgb,
             wvc, bvc, wvg, bvg, wwc, bwc, wwg, bwg,
             wkcx, bkcx, wkcy, bkcy, wkgx, bkgx, wkgy, bkgy):
    def rowspec(n):
        return pl.BlockSpec((1, TQ, n), lambda b, t: (b, t, 0))

    def pairspec(n):
        return pl.BlockSpec((2, 1, TQ, n), lambda b, t: (0, b, t, 0))

    def fullspec(shape):
        nd = len(shape)
        return pl.BlockSpec(shape, lambda b, t: (0,) * nd)

    in_specs = ([rowspec(C)] * 3 + [rowspec(P)] * 4
                + [fullspec(x.shape) for x in (
                    lmg, lmb, lcg, lcb, lgg, lgb,
                    wvc, bvc, wvg, bvg, wwc, bwc, wwg, bwg,
                    wkcx, bkcx, wkcy, bkcy, wkgx, bkgx, wkgy, bkgy)])
    out_shape = [
        jax.ShapeDtypeStruct((2, B, HW, C), jnp.float32),
        jax.ShapeDtypeStruct((2, B, HW, NCW), jnp.float32),
        jax.ShapeDtypeStruct((2, B, HW, NPTS), jnp.int32),
        jax.ShapeDtypeStruct((B, HW, P), jnp.float32),
        jax.ShapeDtypeStruct((B, HW, P), jnp.float32),
        jax.ShapeDtypeStruct((B, HW, P), jnp.float32),
        jax.ShapeDtypeStruct((B, HW, P), jnp.float32),
    ]
    out_specs = [pairspec(C), pairspec(NCW), pairspec(NPTS),
                 rowspec(P), rowspec(P), rowspec(P), rowspec(P)]
    return pl.pallas_call(
        _prep_body, grid=(B, NT),
        in_specs=in_specs, out_specs=out_specs, out_shape=out_shape,
    )(xm, xc, xg, cax, cay, gax, gay,
      lmg, lmb, lcg, lcb, lgg, lgb,
      wvc, bvc, wvg, bvg, wwc, bwc, wwg, bwg,
      wkcx, bkcx, wkcy, bkcy, wkgx, bkgx, wkgy, bkgy)


def _sc_body(tab, idx, cw, out, rows, idx_sc, cw_sc, outb, sem0, sem1):
    cid = lax.axis_index("c")
    sid = lax.axis_index("s")
    wid = sid * NCORES + cid
    base = wid * QPW

    def fire(g, par, sem):
        for qq in range(GRP):
            pltpu.make_async_copy(
                tab.at[idx_sc.at[g * GRP + qq]],
                rows.at[par, qq], sem).start()

    def drain(g, par, sem):
        for qq in range(GRP):
            pltpu.make_async_copy(
                tab.at[idx_sc.at[g * GRP + qq]],
                rows.at[par, qq], sem).wait()

    def compute(g, par):
        @pl.loop(0, GRP)
        def _(qq):
            qa = g * GRP + qq
            cwv = [cw_sc[qa, pl.ds(16 * k, 16)] for k in range(NCW // 16)]
            accs = [jnp.zeros((CG,), jnp.float32) for _ in range(G)]
            for s in range(NPTS):
                for g8 in range(G):
                    col = s * G + g8
                    w = cwv[col // 16][col % 16]
                    accs[g8] = (accs[g8]
                                + rows[par, qq, s, pl.ds(g8 * CG, CG)] * w)
            for g8 in range(G):
                outb[qa, pl.ds(g8 * CG, CG)] = accs[g8]

    @pl.loop(0, NSC)
    def _(sc):
        q0 = base + sc * SCQ
        pltpu.sync_copy(idx.at[pl.ds(q0, SCQ)], idx_sc)
        pltpu.sync_copy(cw.at[pl.ds(q0, SCQ)], cw_sc)
        fire(0, 0, sem0)

        @pl.loop(0, NGRP // 2)
        def _(i):
            g0 = 2 * i
            g1 = 2 * i + 1
            fire(g1, 1, sem1)
            drain(g0, 0, sem0)
            compute(g0, 0)

            @pl.when(i < NGRP // 2 - 1)
            def _():
                fire(g0 + 2, 0, sem0)

            drain(g1, 1, sem1)
            compute(g1, 1)

        pltpu.sync_copy(outb, out.at[pl.ds(q0, SCQ)])


def _sc_agg(tab, idx, cw):
    mesh = plsc.VectorSubcoreMesh(core_axis_name="c", subcore_axis_name="s")
    kern = pl.kernel(
        _sc_body,
        out_type=jax.ShapeDtypeStruct((NQ, C), jnp.float32),
        mesh=mesh,
        scratch_types=[
            pltpu.VMEM((2, GRP, NPTS, C), jnp.float32),
            pltpu.VMEM((SCQ, NPTS), jnp.int32),
            pltpu.VMEM((SCQ, NCW), jnp.float32),
            pltpu.VMEM((SCQ, C), jnp.float32),
            pltpu.SemaphoreType.DMA,
            pltpu.SemaphoreType.DMA,
        ])
    return kern(tab, idx, cw)


def _out_body(oc, og, msk, wo1, wo2, bo, out):
    h = oc[0] * msk[0]
    out[0] = (_dotT(h, wo1[...]) + _dotT(og[0], wo2[...])) + bo[...]


def _tc_out(oc, og, msk, wo1, wo2, bo):
    def rowspec(n):
        return pl.BlockSpec((1, TQ, n), lambda b, t: (b, t, 0))

    def fullspec(shape):
        nd = len(shape)
        return pl.BlockSpec(shape, lambda b, t: (0,) * nd)

    return pl.pallas_call(
        _out_body, grid=(B, NT),
        in_specs=[rowspec(C), rowspec(C), rowspec(1),
                  fullspec(wo1.shape), fullspec(wo2.shape), fullspec(bo.shape)],
        out_specs=rowspec(C),
        out_shape=jax.ShapeDtypeStruct((B, HW, C), jnp.float32),
    )(oc, og, msk, wo1, wo2, bo)


def kernel(mask, match, context, geometric,
           context_anchor_points, geometric_anchor_points,
           ln_match_g, ln_match_b, ln_ctx_g, ln_ctx_b, ln_geo_g, ln_geo_b,
           w_wctx, b_wctx, w_wgeo, b_wgeo, w_vctx, b_vctx, w_vgeo, b_vgeo,
           w_kpctx, b_kpctx, w_kpgeo, b_kpgeo, w_out, b_out):
    f32 = jnp.float32
    xm = match.transpose(0, 2, 3, 1).reshape(B, HW, C)
    xc = context.transpose(0, 2, 3, 1).reshape(B, HW, C)
    xg = geometric.transpose(0, 2, 3, 1).reshape(B, HW, C)
    ca = context_anchor_points.reshape(B, HW, P, 2)
    ga = geometric_anchor_points.reshape(B, HW, P, 2)
    cax, cay = ca[..., 0], ca[..., 1]
    gax, gay = ga[..., 0], ga[..., 1]

    r1 = lambda v: v.reshape(1, -1).astype(f32)
    outs = _tc_prep(
        xm, xc, xg, cax, cay, gax, gay,
        r1(ln_match_g), r1(ln_match_b), r1(ln_ctx_g), r1(ln_ctx_b),
        r1(ln_geo_g), r1(ln_geo_b),
        w_vctx, r1(b_vctx), w_vgeo, r1(b_vgeo),
        w_wctx, r1(b_wctx), w_wgeo, r1(b_wgeo),
        w_kpctx[0::2], r1(b_kpctx[0::2]), w_kpctx[1::2], r1(b_kpctx[1::2]),
        w_kpgeo[0::2], r1(b_kpgeo[0::2]), w_kpgeo[1::2], r1(b_kpgeo[1::2]))
    (vtab, cw, idx, kxc, kyc, kxg, kyg) = outs

    o = _sc_agg(vtab.reshape(NQ, C), idx.reshape(NQ, NPTS),
                cw.reshape(NQ, NCW))
    oc = o[:B * HW].reshape(B, HW, C)
    og = o[B * HW:].reshape(B, HW, C)

    out_rows = _tc_out(oc, og, mask.reshape(B, HW, 1),
                       w_out[:, :C], w_out[:, C:], r1(b_out))

    out = out_rows.transpose(0, 2, 1).reshape(B, C, H, W)
    kc = jnp.stack([kxc, kyc], axis=-1).reshape(B, H, W, P, 2)
    kg = jnp.stack([kxg, kyg], axis=-1).reshape(B, H, W, P, 2)
    return out, kc, kg


# speedup vs baseline: 51.7911x; 1.4202x over previous
import dataclasses
import functools

import jax
import jax.numpy as jnp
from jax import lax
from jax.experimental import pallas as pl
from jax.experimental.pallas import tpu as pltpu
from jax.experimental.pallas import tpu_sc as plsc

B = 2
C = 128
H = 96
W = 96
P = 9
G = 8
CG = C // G
HW = H * W
NPTS = 4 * P
NCW = NPTS * G
NQB = 2 * HW

TQ = 512
NT = HW // TQ

NCORES = 2
NSUB = 16
NW = NCORES * NSUB
QPW = NQB // NW
SCQ = 64
NSC = QPW // SCQ
GRP = 8
NGRP = SCQ // GRP

_HIGH = jax.lax.Precision.HIGHEST


def _dotT(a, w):
    return lax.dot_general(a, w, (((1,), (1,)), ((), ())),
                           preferred_element_type=jnp.float32)


def _dot01(a, m01):
    return lax.dot_general(a, m01, (((1,), (0,)), ((), ())),
                           preferred_element_type=jnp.float32)


def _ln(x, g, b):
    mu = jnp.mean(x, axis=-1, keepdims=True)
    var = jnp.mean((x - mu) ** 2, axis=-1, keepdims=True)
    return (x - mu) * lax.rsqrt(var + 1e-5) * g + b


def _prep_body(xm, xc, xg, ax2, ay2,
               lmg, lmb, lcg, lcb, lgg, lgb,
               wvc, bvc, wvg, bvg, wwc, bwc, wwg, bwg,
               wkx2, bkx2, wky2, bky2,
               o_vtab, o_cw, o_idx, o_kx2, o_ky2):
    mn = _ln(xm[0], lmg[...], lmb[...])
    cn = _ln(xc[0], lcg[...], lcb[...])
    gn = _ln(xg[0], lgg[...], lgb[...])

    o_vtab[0] = _dotT(cn, wvc[...]) + bvc[...]
    o_vtab[1] = _dotT(gn, wvg[...]) + bvg[...]

    i32 = jnp.int32
    f32 = jnp.float32
    rep = (lax.broadcasted_iota(i32, (NPTS, NCW), 1) // G
           == lax.broadcasted_iota(i32, (NPTS, NCW), 0)).astype(f32)
    s8 = (lax.broadcasted_iota(i32, (P * G, G), 0) % G
          == lax.broadcasted_iota(i32, (P * G, G), 1)).astype(f32)
    s8t = (lax.broadcasted_iota(i32, (G, P * G), 1) % G
           == lax.broadcasted_iota(i32, (G, P * G), 0)).astype(f32)
    t4r = lax.broadcasted_iota(i32, (P * G, NCW), 0)
    t4c = lax.broadcasted_iota(i32, (P * G, NCW), 1)
    t4 = ((t4c % G == t4r % G)
          & ((t4c // G) % P == t4r // G)).astype(f32)

    kx2 = ax2[0] + _dotT(mn, wkx2[...]) + bkx2[...]
    ky2 = ay2[0] + _dotT(mn, wky2[...]) + bky2[...]
    o_kx2[...] = kx2
    o_ky2[...] = ky2

    x = kx2 * W - 0.5
    y = ky2 * H - 0.5
    x0 = jnp.floor(x)
    y0 = jnp.floor(y)
    wx1 = x - x0
    wx0 = 1.0 - wx1
    wy1 = y - y0
    wy0 = 1.0 - wy1
    lane18 = lax.broadcasted_iota(i32, (1, 2 * P), 1)
    base18 = jnp.where(lane18 < P, 0, HW)
    idx18 = []
    bw18 = []
    for dx, dy, wxs, wys in ((0, 0, wx0, wy0), (1, 0, wx1, wy0),
                             (0, 1, wx0, wy1), (1, 1, wx1, wy1)):
        xi = x0 + dx
        yi = y0 + dy
        valid = ((xi >= 0.0) & (xi <= W - 1.0)
                 & (yi >= 0.0) & (yi <= H - 1.0))
        xcl = jnp.clip(xi, 0.0, W - 1.0).astype(i32)
        ycl = jnp.clip(yi, 0.0, H - 1.0).astype(i32)
        idx18.append(ycl * W + xcl + base18)
        bw18.append(wxs * wys * valid.astype(f32))

    for tsel, (ww, bw) in enumerate(((wwc, bwc), (wwg, bwg))):
        sl = slice(0, P) if tsel == 0 else slice(P, 2 * P)
        o_idx[tsel] = jnp.concatenate([c[:, sl] for c in idx18], axis=1)
        bwm = jnp.concatenate([c[:, sl] for c in bw18], axis=1)

        logits = _dotT(mn, ww[...]) + bw[...]
        m = jnp.max(logits, axis=-1, keepdims=True)
        e = jnp.exp(logits - m)
        inv = 1.0 / _dot01(e, s8)
        gw = e * _dot01(inv, s8t)
        o_cw[tsel] = _dot01(bwm, rep) * _dot01(gw, t4)


def _tc_prep(b, xm, xc, xg, ax2, ay2,
             lmg, lmb, lcg, lcb, lgg, lgb,
             wvc, bvc, wvg, bvg, wwc, bwc, wwg, bwg,
             wkx2, bkx2, wky2, bky2):
    def rmspec(n):
        return pl.BlockSpec((1, TQ, n), lambda t: (b, t, 0))

    def pairspec(n):
        return pl.BlockSpec((2, TQ, n), lambda t: (0, t, 0))

    def outspec(n):
        return pl.BlockSpec((TQ, n), lambda t: (t, 0))

    def fullspec(shape):
        nd = len(shape)
        return pl.BlockSpec(shape, lambda t: (0,) * nd)

    in_specs = ([rmspec(C)] * 3 + [rmspec(2 * P)] * 2
                + [fullspec(x.shape) for x in (
                    lmg, lmb, lcg, lcb, lgg, lgb,
                    wvc, bvc, wvg, bvg, wwc, bwc, wwg, bwg,
                    wkx2, bkx2, wky2, bky2)])
    out_shape = [
        jax.ShapeDtypeStruct((2, HW, C), jnp.float32),
        jax.ShapeDtypeStruct((2, HW, NCW), jnp.float32),
        jax.ShapeDtypeStruct((2, HW, NPTS), jnp.int32),
        jax.ShapeDtypeStruct((HW, 2 * P), jnp.float32),
        jax.ShapeDtypeStruct((HW, 2 * P), jnp.float32),
    ]
    out_specs = [pairspec(C), pairspec(NCW), pairspec(NPTS),
                 outspec(2 * P), outspec(2 * P)]
    body = _prep_body

    def bodyfn(*refs):
        return body(*refs)

    return pl.pallas_call(
        bodyfn, grid=(NT,),
        in_specs=in_specs, out_specs=out_specs, out_shape=out_shape,
    )(xm, xc, xg, ax2, ay2,
      lmg, lmb, lcg, lcb, lgg, lgb,
      wvc, bvc, wvg, bvg, wwc, bwc, wwg, bwg,
      wkx2, bkx2, wky2, bky2)


def _sc_body(tab, idx, cw, out, rows, idx_sc, cw_sc, outb, sem0, sem1):
    cid = lax.axis_index("c")
    sid = lax.axis_index("s")
    wid = sid * NCORES + cid
    base = wid * QPW

    def fire(g, par, sem):
        for qq in range(GRP):
            pltpu.make_async_copy(
                tab.at[idx_sc.at[g * GRP + qq]],
                rows.at[par, qq], sem).start()

    def drain(g, par, sem):
        for qq in range(GRP):
            pltpu.make_async_copy(
                tab.at[idx_sc.at[g * GRP + qq]],
                rows.at[par, qq], sem).wait()

    lane_ids = [jnp.full((16,), j, jnp.int32) for j in range(16)]

    def compute(g, par):
        @pl.loop(0, GRP)
        def _(qq):
            qa = g * GRP + qq
            cwv = [cw_sc[qa, pl.ds(16 * k, 16)] for k in range(NCW // 16)]
            accs = [jnp.zeros((CG,), jnp.float32) for _ in range(G)]
            for s in range(NPTS):
                for g8 in range(G):
                    col = s * G + g8
                    w = cwv[col // 16].at[lane_ids[col % 16]].get(
                        mode="promise_in_bounds")
                    accs[g8] = (accs[g8]
                                + rows[par, qq, s, pl.ds(g8 * CG, CG)] * w)
            for g8 in range(G):
                outb[qa, pl.ds(g8 * CG, CG)] = accs[g8]

    @pl.loop(0, NSC)
    def _(sc):
        q0 = base + sc * SCQ
        pltpu.sync_copy(idx.at[pl.ds(q0, SCQ)], idx_sc)
        pltpu.sync_copy(cw.at[pl.ds(q0, SCQ)], cw_sc)
        fire(0, 0, sem0)

        @pl.loop(0, NGRP // 2)
        def _(i):
            g0 = 2 * i
            g1 = 2 * i + 1
            fire(g1, 1, sem1)
            drain(g0, 0, sem0)
            compute(g0, 0)

            @pl.when(i < NGRP // 2 - 1)
            def _():
                fire(g0 + 2, 0, sem0)

            drain(g1, 1, sem1)
            compute(g1, 1)

        pltpu.sync_copy(outb, out.at[pl.ds(q0, SCQ)])


def _sc_agg(tab, idx, cw):
    mesh = plsc.VectorSubcoreMesh(core_axis_name="c", subcore_axis_name="s")
    kern = pl.kernel(
        _sc_body,
        out_type=jax.ShapeDtypeStruct((NQB, C), jnp.float32),
        mesh=mesh,
        scratch_types=[
            pltpu.VMEM((2, GRP, NPTS, C), jnp.float32),
            pltpu.VMEM((SCQ, NPTS), jnp.int32),
            pltpu.VMEM((SCQ, NCW), jnp.float32),
            pltpu.VMEM((SCQ, C), jnp.float32),
            pltpu.SemaphoreType.DMA,
            pltpu.SemaphoreType.DMA,
        ])
    return kern(tab, idx, cw)


def _out_body(oc, og, msk, wo1, wo2, bo, out):
    h = oc[...] * msk[0]
    res = (_dotT(h, wo1[...]) + _dotT(og[...], wo2[...])) + bo[...]
    out[...] = jnp.transpose(res)


def _tc_out(b, o, msk, wo1, wo2, bo):
    def fullspec(shape):
        nd = len(shape)
        return pl.BlockSpec(shape, lambda t: (0,) * nd)

    return pl.pallas_call(
        _out_body, grid=(NT,),
        in_specs=[pl.BlockSpec((TQ, C), lambda t: (t, 0)),
                  pl.BlockSpec((TQ, C), lambda t: (NT + t, 0)),
                  pl.BlockSpec((1, TQ, 1), lambda t: (b, t, 0)),
                  fullspec(wo1.shape), fullspec(wo2.shape), fullspec(bo.shape)],
        out_specs=pl.BlockSpec((C, TQ), lambda t: (0, t)),
        out_shape=jax.ShapeDtypeStruct((C, HW), jnp.float32),
    )(o, o, msk, wo1, wo2, bo)


def kernel(mask, match, context, geometric,
           context_anchor_points, geometric_anchor_points,
           ln_match_g, ln_match_b, ln_ctx_g, ln_ctx_b, ln_geo_g, ln_geo_b,
           w_wctx, b_wctx, w_wgeo, b_wgeo, w_vctx, b_vctx, w_vgeo, b_vgeo,
           w_kpctx, b_kpctx, w_kpgeo, b_kpgeo, w_out, b_out):
    f32 = jnp.float32
    xm = match.transpose(0, 2, 3, 1).reshape(B, HW, C)
    xc = context.transpose(0, 2, 3, 1).reshape(B, HW, C)
    xg = geometric.transpose(0, 2, 3, 1).reshape(B, HW, C)
    ca = context_anchor_points.reshape(B, HW, P, 2)
    ga = geometric_anchor_points.reshape(B, HW, P, 2)
    ax2 = jnp.concatenate([ca[..., 0], ga[..., 0]], axis=-1)
    ay2 = jnp.concatenate([ca[..., 1], ga[..., 1]], axis=-1)
    mask_r = mask.reshape(B, HW, 1)

    r1 = lambda v: v.reshape(1, -1).astype(f32)
    wkx2 = jnp.concatenate([w_kpctx[0::2], w_kpgeo[0::2]], axis=0)
    wky2 = jnp.concatenate([w_kpctx[1::2], w_kpgeo[1::2]], axis=0)
    bkx2 = jnp.concatenate([b_kpctx[0::2], b_kpgeo[0::2]]).reshape(1, -1)
    bky2 = jnp.concatenate([b_kpctx[1::2], b_kpgeo[1::2]]).reshape(1, -1)
    wo1, wo2 = w_out[:, :C], w_out[:, C:]

    outs = []
    kxs = []
    kys = []
    for b in range(B):
        vtab, cw, idx, kx2, ky2 = _tc_prep(
            b, xm, xc, xg, ax2, ay2,
            r1(ln_match_g), r1(ln_match_b), r1(ln_ctx_g), r1(ln_ctx_b),
            r1(ln_geo_g), r1(ln_geo_b),
            w_vctx, r1(b_vctx), w_vgeo, r1(b_vgeo),
            w_wctx, r1(b_wctx), w_wgeo, r1(b_wgeo),
            wkx2, bkx2, wky2, bky2)
        kxs.append(kx2)
        kys.append(ky2)

        o = _sc_agg(vtab.reshape(NQB, C), idx.reshape(NQB, NPTS),
                    cw.reshape(NQB, NCW))
        outs.append(_tc_out(b, o, mask_r, wo1, wo2, r1(b_out)))

    out = jnp.stack(outs).reshape(B, C, H, W)
    kx2 = jnp.stack(kxs)
    ky2 = jnp.stack(kys)
    kc = jnp.stack([kx2[..., :P], ky2[..., :P]], axis=-1).reshape(
        B, H, W, P, 2)
    kg = jnp.stack([kx2[..., P:], ky2[..., P:]], axis=-1).reshape(
        B, H, W, P, 2)
    return out, kc, kg
